# baseline (speedup 1.0000x reference)
import math
import functools

import jax
import jax.numpy as jnp
from jax.experimental import pallas as pl
from jax.experimental.pallas import tpu as pltpu

_BLOCKSIZE = 32
_BATCH_TILE = 2


def _attn_kernel(x_ref, wqkv_ref, bqkv_ref, wproj_ref, bproj_ref, o_ref,
                 *, num_heads, head_dim, blocksize, batch_tile):
    N = x_ref.shape[1]
    C = num_heads * head_dim
    nblk = N // blocksize

    blk_row = jax.lax.broadcasted_iota(jnp.int32, (N, nblk), 0) // blocksize
    blk_col = jax.lax.broadcasted_iota(jnp.int32, (N, nblk), 1)
    a_onehot = (blk_row == blk_col).astype(jnp.bfloat16)

    ones_pad = jnp.ones((N, head_dim), jnp.bfloat16)

    qkvs = []
    for e in range(batch_tile):
        x = x_ref[e].astype(jnp.bfloat16)
        qkv = jnp.dot(x, wqkv_ref[...], preferred_element_type=jnp.float32)
        qkvs.append((qkv + bqkv_ref[...]).astype(jnp.bfloat16))

    outs_t = [[] for _ in range(batch_tile)]
    for h in range(num_heads):
        lo = h * head_dim
        hi = lo + head_dim
        for e in range(batch_tile):
            qkv = qkvs[e]
            q_aug = jnp.concatenate([qkv[:, lo:hi], a_onehot], axis=1)
            k_aug = jnp.concatenate([qkv[:, C + lo:C + hi], a_onehot], axis=1)
            v = qkv[:, 2 * C + lo:2 * C + hi]

            s_t = jax.lax.dot_general(
                k_aug, q_aug, (((1,), (1,)), ((), ())),
                preferred_element_type=jnp.float32)
            p_t = jnp.exp(s_t).astype(jnp.bfloat16)
            v_aug = jnp.concatenate([v, ones_pad], axis=1)
            o_full = jax.lax.dot_general(
                v_aug, p_t, (((0,), (0,)), ((), ())),
                preferred_element_type=jnp.float32)
            o_ht = (o_full[:head_dim]
                    * pl.reciprocal(o_full[head_dim:2 * head_dim]))
            outs_t[e].append(o_ht.astype(jnp.bfloat16))

    for e in range(batch_tile):
        attn_t = jnp.concatenate(outs_t[e], axis=0)
        out = jax.lax.dot_general(
            attn_t, wproj_ref[...], (((0,), (0,)), ((), ())),
            preferred_element_type=jnp.float32)
        o_ref[e] = out + bproj_ref[...]


def kernel(x, w_qkv, b_qkv, w_proj, b_proj):
    B, N, C = x.shape
    num_heads = 12
    head_dim = C // num_heads
    scale = 1.0 / math.sqrt(head_dim)

    scale_vec = jnp.concatenate(
        [jnp.full((C,), scale, jnp.float32),
         jnp.ones((2 * C,), jnp.float32)])
    wqkv_bf = (w_qkv * scale_vec[None, :]).astype(jnp.bfloat16)
    bqkv_s = b_qkv * scale_vec[None, :]
    wproj_bf = w_proj.astype(jnp.bfloat16)

    bt = _BATCH_TILE

    body = functools.partial(
        _attn_kernel, num_heads=num_heads, head_dim=head_dim,
        blocksize=_BLOCKSIZE, batch_tile=bt)

    return pl.pallas_call(
        body,
        out_shape=jax.ShapeDtypeStruct((B, N, C), jnp.float32),
        grid=(B // bt,),
        in_specs=[
            pl.BlockSpec((bt, N, C), lambda b: (b, 0, 0)),
            pl.BlockSpec((C, 3 * C), lambda b: (0, 0)),
            pl.BlockSpec((1, 3 * C), lambda b: (0, 0)),
            pl.BlockSpec((C, C), lambda b: (0, 0)),
            pl.BlockSpec((1, C), lambda b: (0, 0)),
        ],
        out_specs=pl.BlockSpec((bt, N, C), lambda b: (b, 0, 0)),
        compiler_params=pltpu.CompilerParams(
            dimension_semantics=("parallel",)),
    )(x, wqkv_bf, bqkv_s, wproj_bf, b_proj)


# speedup vs baseline: 1.1110x; 1.1110x over previous
import math
import functools

import jax
import jax.numpy as jnp
from jax.experimental import pallas as pl
from jax.experimental.pallas import tpu as pltpu

_BLOCKSIZE = 32


def _attn_kernel(x_ref, wqkv_ref, bqkv_ref, wproj_ref, bproj_ref, o_ref,
                 *, num_heads, head_dim, blocksize):
    x = x_ref[0].astype(jnp.bfloat16)
    N = x.shape[0]
    C = num_heads * head_dim

    qkv = jnp.dot(x, wqkv_ref[...], preferred_element_type=jnp.float32)
    qkv = (qkv + bqkv_ref[...]).astype(jnp.bfloat16)

    row = jax.lax.broadcasted_iota(jnp.int32, (N, N), 0) // blocksize
    col = jax.lax.broadcasted_iota(jnp.int32, (N, N), 1) // blocksize
    bias = (row == col).astype(jnp.float32)

    ones_pad = jnp.ones((N, head_dim), jnp.bfloat16)

    outs = []
    for h in range(num_heads):
        lo = h * head_dim
        hi = lo + head_dim
        q = qkv[:, lo:hi]
        k = qkv[:, C + lo:C + hi]
        v = qkv[:, 2 * C + lo:2 * C + hi]

        s = jax.lax.dot_general(
            q, k, (((1,), (1,)), ((), ())),
            preferred_element_type=jnp.float32) + bias
        p = jnp.exp(s).astype(jnp.bfloat16)
        v_aug = jnp.concatenate([v, ones_pad], axis=1)
        o_full = jnp.dot(p, v_aug, preferred_element_type=jnp.float32)
        o_h = (o_full[:, :head_dim]
               * pl.reciprocal(o_full[:, head_dim:2 * head_dim]))
        outs.append(o_h.astype(jnp.bfloat16))

    attn = jnp.concatenate(outs, axis=1)
    out = jnp.dot(attn, wproj_ref[...], preferred_element_type=jnp.float32)
    o_ref[0] = out + bproj_ref[...]


def kernel(x, w_qkv, b_qkv, w_proj, b_proj):
    B, N, C = x.shape
    num_heads = 12
    head_dim = C // num_heads
    scale = 1.0 / math.sqrt(head_dim)

    scale_vec = jnp.concatenate(
        [jnp.full((C,), scale, jnp.float32),
         jnp.ones((2 * C,), jnp.float32)])
    wqkv_bf = (w_qkv * scale_vec[None, :]).astype(jnp.bfloat16)
    bqkv_s = b_qkv * scale_vec[None, :]
    wproj_bf = w_proj.astype(jnp.bfloat16)

    body = functools.partial(
        _attn_kernel, num_heads=num_heads, head_dim=head_dim,
        blocksize=_BLOCKSIZE)

    return pl.pallas_call(
        body,
        out_shape=jax.ShapeDtypeStruct((B, N, C), jnp.float32),
        grid=(B,),
        in_specs=[
            pl.BlockSpec((1, N, C), lambda b: (b, 0, 0)),
            pl.BlockSpec((C, 3 * C), lambda b: (0, 0)),
            pl.BlockSpec((1, 3 * C), lambda b: (0, 0)),
            pl.BlockSpec((C, C), lambda b: (0, 0)),
            pl.BlockSpec((1, C), lambda b: (0, 0)),
        ],
        out_specs=pl.BlockSpec((1, N, C), lambda b: (b, 0, 0)),
        compiler_params=pltpu.CompilerParams(
            dimension_semantics=("parallel",)),
    )(x, wqkv_bf, bqkv_s, wproj_bf, b_proj)


# speedup vs baseline: 1.1133x; 1.0021x over previous
import math
import functools

import jax
import jax.numpy as jnp
from jax.experimental import pallas as pl
from jax.experimental.pallas import tpu as pltpu

_BLOCKSIZE = 32


def _attn_kernel(x_ref, wqkv_ref, bqkv_ref, wproj_ref, bproj_ref, o_ref,
                 *, num_heads, head_dim, blocksize):
    x = x_ref[0].astype(jnp.bfloat16)
    N = x.shape[0]
    C = num_heads * head_dim

    qkv = jnp.dot(x, wqkv_ref[...], preferred_element_type=jnp.float32)
    qkv = (qkv + bqkv_ref[...]).astype(jnp.bfloat16)

    row = jax.lax.broadcasted_iota(jnp.int32, (N, N), 0) // blocksize
    col = jax.lax.broadcasted_iota(jnp.int32, (N, N), 1) // blocksize
    blk = (row == col).astype(jnp.float32)
    e_mask = (1.0 + (math.e - 1.0) * blk).astype(jnp.bfloat16)

    ones_pad = jnp.ones((N, head_dim), jnp.bfloat16)

    outs = []
    for h in range(num_heads):
        lo = h * head_dim
        hi = lo + head_dim
        q = qkv[:, lo:hi]
        k = qkv[:, C + lo:C + hi]
        v = qkv[:, 2 * C + lo:2 * C + hi]

        s = jax.lax.dot_general(
            q, k, (((1,), (1,)), ((), ())),
            preferred_element_type=jnp.float32)
        p = jnp.exp(s).astype(jnp.bfloat16) * e_mask
        v_aug = jnp.concatenate([v, ones_pad], axis=1)
        o_full = jnp.dot(p, v_aug, preferred_element_type=jnp.float32)
        o_h = (o_full[:, :head_dim]
               * pl.reciprocal(o_full[:, head_dim:2 * head_dim]))
        outs.append(o_h.astype(jnp.bfloat16))

    attn = jnp.concatenate(outs, axis=1)
    out = jnp.dot(attn, wproj_ref[...], preferred_element_type=jnp.float32)
    o_ref[0] = out + bproj_ref[...]


def kernel(x, w_qkv, b_qkv, w_proj, b_proj):
    B, N, C = x.shape
    num_heads = 12
    head_dim = C // num_heads
    scale = 1.0 / math.sqrt(head_dim)

    scale_vec = jnp.concatenate(
        [jnp.full((C,), scale, jnp.float32),
         jnp.ones((2 * C,), jnp.float32)])
    wqkv_bf = (w_qkv * scale_vec[None, :]).astype(jnp.bfloat16)
    bqkv_s = b_qkv * scale_vec[None, :]
    wproj_bf = w_proj.astype(jnp.bfloat16)

    body = functools.partial(
        _attn_kernel, num_heads=num_heads, head_dim=head_dim,
        blocksize=_BLOCKSIZE)

    return pl.pallas_call(
        body,
        out_shape=jax.ShapeDtypeStruct((B, N, C), jnp.float32),
        grid=(B,),
        in_specs=[
            pl.BlockSpec((1, N, C), lambda b: (b, 0, 0)),
            pl.BlockSpec((C, 3 * C), lambda b: (0, 0)),
            pl.BlockSpec((1, 3 * C), lambda b: (0, 0)),
            pl.BlockSpec((C, C), lambda b: (0, 0)),
            pl.BlockSpec((1, C), lambda b: (0, 0)),
        ],
        out_specs=pl.BlockSpec((1, N, C), lambda b: (b, 0, 0)),
        compiler_params=pltpu.CompilerParams(
            dimension_semantics=("parallel",)),
    )(x, wqkv_bf, bqkv_s, wproj_bf, b_proj)


# speedup vs baseline: 1.1563x; 1.0386x over previous
import math
import functools

import jax
import jax.numpy as jnp
from jax.experimental import pallas as pl
from jax.experimental.pallas import tpu as pltpu

_BLOCKSIZE = 32
_BATCH_TILE = 2


def _attn_kernel(x_ref, wqkv_ref, bqkv_ref, wproj_ref, bproj_ref, o_ref,
                 *, num_heads, head_dim, blocksize, batch_tile):
    N = x_ref.shape[1]
    C = num_heads * head_dim

    row = jax.lax.broadcasted_iota(jnp.int32, (N, N), 0) // blocksize
    col = jax.lax.broadcasted_iota(jnp.int32, (N, N), 1) // blocksize
    blk = (row == col).astype(jnp.float32)
    e_mask = (1.0 + (math.e - 1.0) * blk).astype(jnp.bfloat16)

    ones_pad = jnp.ones((N, head_dim), jnp.bfloat16)

    for e in range(batch_tile):
        x = x_ref[e].astype(jnp.bfloat16)
        qkv = jnp.dot(x, wqkv_ref[...], preferred_element_type=jnp.float32)
        qkv = (qkv + bqkv_ref[...]).astype(jnp.bfloat16)

        outs = []
        for h in range(num_heads):
            lo = h * head_dim
            hi = lo + head_dim
            q = qkv[:, lo:hi]
            k = qkv[:, C + lo:C + hi]
            v = qkv[:, 2 * C + lo:2 * C + hi]

            s = jax.lax.dot_general(
                q, k, (((1,), (1,)), ((), ())),
                preferred_element_type=jnp.float32)
            p = jnp.exp(s).astype(jnp.bfloat16) * e_mask
            v_aug = jnp.concatenate([v, ones_pad], axis=1)
            o_full = jnp.dot(p, v_aug, preferred_element_type=jnp.float32)
            o_h = (o_full[:, :head_dim]
                   * pl.reciprocal(o_full[:, head_dim:2 * head_dim]))
            outs.append(o_h.astype(jnp.bfloat16))

        attn = jnp.concatenate(outs, axis=1)
        out = jnp.dot(attn, wproj_ref[...],
                      preferred_element_type=jnp.float32)
        o_ref[e] = out + bproj_ref[...]


def kernel(x, w_qkv, b_qkv, w_proj, b_proj):
    B, N, C = x.shape
    num_heads = 12
    head_dim = C // num_heads
    scale = 1.0 / math.sqrt(head_dim)

    scale_vec = jnp.concatenate(
        [jnp.full((C,), scale, jnp.float32),
         jnp.ones((2 * C,), jnp.float32)])
    wqkv_bf = (w_qkv * scale_vec[None, :]).astype(jnp.bfloat16)
    bqkv_s = b_qkv * scale_vec[None, :]
    wproj_bf = w_proj.astype(jnp.bfloat16)

    bt = _BATCH_TILE
    body = functools.partial(
        _attn_kernel, num_heads=num_heads, head_dim=head_dim,
        blocksize=_BLOCKSIZE, batch_tile=bt)

    return pl.pallas_call(
        body,
        out_shape=jax.ShapeDtypeStruct((B, N, C), jnp.float32),
        grid=(B // bt,),
        in_specs=[
            pl.BlockSpec((bt, N, C), lambda b: (b, 0, 0)),
            pl.BlockSpec((C, 3 * C), lambda b: (0, 0)),
            pl.BlockSpec((1, 3 * C), lambda b: (0, 0)),
            pl.BlockSpec((C, C), lambda b: (0, 0)),
            pl.BlockSpec((1, C), lambda b: (0, 0)),
        ],
        out_specs=pl.BlockSpec((bt, N, C), lambda b: (b, 0, 0)),
        compiler_params=pltpu.CompilerParams(
            dimension_semantics=("parallel",)),
    )(x, wqkv_bf, bqkv_s, wproj_bf, b_proj)


# speedup vs baseline: 1.1735x; 1.0149x over previous
import math
import functools

import jax
import jax.numpy as jnp
from jax.experimental import pallas as pl
from jax.experimental.pallas import tpu as pltpu

_BLOCKSIZE = 32
_BATCH_TILE = 4


def _attn_kernel(x_ref, wqkv_ref, bqkv_ref, wproj_ref, bproj_ref, o_ref,
                 *, num_heads, head_dim, blocksize, batch_tile):
    N = x_ref.shape[1]
    C = num_heads * head_dim

    row = jax.lax.broadcasted_iota(jnp.int32, (N, N), 0) // blocksize
    col = jax.lax.broadcasted_iota(jnp.int32, (N, N), 1) // blocksize
    blk = (row == col).astype(jnp.float32)
    e_mask = (1.0 + (math.e - 1.0) * blk).astype(jnp.bfloat16)

    ones_pad = jnp.ones((N, head_dim), jnp.bfloat16)

    for e in range(batch_tile):
        x = x_ref[e].astype(jnp.bfloat16)
        qkv = jnp.dot(x, wqkv_ref[...], preferred_element_type=jnp.float32)
        qkv = (qkv + bqkv_ref[...]).astype(jnp.bfloat16)

        outs = []
        for h in range(num_heads):
            lo = h * head_dim
            hi = lo + head_dim
            q = qkv[:, lo:hi]
            k = qkv[:, C + lo:C + hi]
            v = qkv[:, 2 * C + lo:2 * C + hi]

            s = jax.lax.dot_general(
                q, k, (((1,), (1,)), ((), ())),
                preferred_element_type=jnp.float32)
            p = jnp.exp(s).astype(jnp.bfloat16) * e_mask
            v_aug = jnp.concatenate([v, ones_pad], axis=1)
            o_full = jnp.dot(p, v_aug, preferred_element_type=jnp.float32)
            o_h = (o_full[:, :head_dim]
                   * pl.reciprocal(o_full[:, head_dim:2 * head_dim]))
            outs.append(o_h.astype(jnp.bfloat16))

        attn = jnp.concatenate(outs, axis=1)
        out = jnp.dot(attn, wproj_ref[...],
                      preferred_element_type=jnp.float32)
        o_ref[e] = out + bproj_ref[...]


def kernel(x, w_qkv, b_qkv, w_proj, b_proj):
    B, N, C = x.shape
    num_heads = 12
    head_dim = C // num_heads
    scale = 1.0 / math.sqrt(head_dim)

    scale_vec = jnp.concatenate(
        [jnp.full((C,), scale, jnp.float32),
         jnp.ones((2 * C,), jnp.float32)])
    wqkv_bf = (w_qkv * scale_vec[None, :]).astype(jnp.bfloat16)
    bqkv_s = b_qkv * scale_vec[None, :]
    wproj_bf = w_proj.astype(jnp.bfloat16)

    bt = _BATCH_TILE
    body = functools.partial(
        _attn_kernel, num_heads=num_heads, head_dim=head_dim,
        blocksize=_BLOCKSIZE, batch_tile=bt)

    return pl.pallas_call(
        body,
        out_shape=jax.ShapeDtypeStruct((B, N, C), jnp.float32),
        grid=(B // bt,),
        in_specs=[
            pl.BlockSpec((bt, N, C), lambda b: (b, 0, 0)),
            pl.BlockSpec((C, 3 * C), lambda b: (0, 0)),
            pl.BlockSpec((1, 3 * C), lambda b: (0, 0)),
            pl.BlockSpec((C, C), lambda b: (0, 0)),
            pl.BlockSpec((1, C), lambda b: (0, 0)),
        ],
        out_specs=pl.BlockSpec((bt, N, C), lambda b: (b, 0, 0)),
        compiler_params=pltpu.CompilerParams(
            dimension_semantics=("parallel",)),
    )(x, wqkv_bf, bqkv_s, wproj_bf, b_proj)
